# baseline (device time: 11654 ns/iter reference)
import jax
import jax.numpy as jnp
from jax import lax
from jax.experimental import pallas as pl
from jax.experimental.pallas import tpu as pltpu

N_DEV = 4
NUM_CHUNKS = 4
DIRECT = 1
NB = NUM_CHUNKS - DIRECT
WIRE_DTYPE = jnp.bfloat16


def kernel(x, W1, W2):
    m, _ = x.shape
    n = W2.shape[1]
    mc = m // NUM_CHUNKS

    def body(x_ref, w1_ref, w2_ref, out_ref, send_ref, comm_ref,
             send_sems, recv_sems):
        my_pos = lax.axis_index("i")
        peers = [my_pos ^ 1, 3 - my_pos, my_pos ^ 2]

        barrier_sem = pltpu.get_barrier_semaphore()
        for nbr in peers:
            pl.semaphore_signal(
                barrier_sem, inc=1,
                device_id=(nbr,), device_id_type=pl.DeviceIdType.MESH,
            )

        def rdma_to(peer_k, slot, src_slot):
            return pltpu.make_async_remote_copy(
                src_ref=send_ref.at[src_slot],
                dst_ref=comm_ref.at[slot],
                send_sem=send_sems.at[slot],
                recv_sem=recv_sems.at[slot],
                device_id=(peers[peer_k],),
                device_id_type=pl.DeviceIdType.MESH,
            )

        rdmas = {}
        for c in range(NUM_CHUNKS):
            rows = pl.ds(c * mc, mc)
            hidden = jnp.maximum(
                jnp.dot(x_ref[rows, :], w1_ref[:, :],
                        preferred_element_type=jnp.float32),
                0.0,
            )
            p = jnp.dot(hidden, w2_ref[:, :],
                        preferred_element_type=jnp.float32)
            out_ref[rows, :] = p
            if c == 0:
                pl.semaphore_wait(barrier_sem, len(peers))
            if c < NB:
                send_ref[c, :, :] = p.astype(WIRE_DTYPE)
                rdmas[("b", 0, c)] = rdma_to(0, c, c)
                rdmas[("b", 0, c)].start()
            else:
                d = c - NB
                src = 2 * NB + d * 3
                send_ref[src, :, :] = p.astype(WIRE_DTYPE)
                for k in range(3):
                    rdmas[("d", d, k)] = rdma_to(k, src + k, src)
                    rdmas[("d", d, k)].start()

        for c in range(NB):
            rows = pl.ds(c * mc, mc)
            rdmas[("b", 0, c)].wait()
            acc = out_ref[rows, :] + comm_ref[c, :, :].astype(jnp.float32)
            out_ref[rows, :] = acc
            send_ref[NB + c, :, :] = acc.astype(WIRE_DTYPE)
            rdmas[("b", 1, c)] = rdma_to(1, NB + c, NB + c)
            rdmas[("b", 1, c)].start()

        for c in range(NB):
            rows = pl.ds(c * mc, mc)
            rdmas[("b", 1, c)].wait()
            out_ref[rows, :] = (
                out_ref[rows, :] + comm_ref[NB + c, :, :].astype(jnp.float32)
            )

        for d in range(DIRECT):
            c = NB + d
            rows = pl.ds(c * mc, mc)
            for k in range(3):
                rdmas[("d", d, k)].wait_recv()
            base = 2 * NB + d * 3
            out_ref[rows, :] = (
                out_ref[rows, :]
                + comm_ref[base + 0, :, :].astype(jnp.float32)
                + comm_ref[base + 1, :, :].astype(jnp.float32)
                + comm_ref[base + 2, :, :].astype(jnp.float32)
            )

        for d in range(DIRECT):
            for k in range(3):
                rdmas[("d", d, k)].wait_send()

    n_slots = 2 * NB + 3 * DIRECT
    return pl.pallas_call(
        body,
        out_shape=jax.ShapeDtypeStruct((m, n), jnp.float32),
        in_specs=[
            pl.BlockSpec(memory_space=pltpu.VMEM),
            pl.BlockSpec(memory_space=pltpu.VMEM),
            pl.BlockSpec(memory_space=pltpu.VMEM),
        ],
        out_specs=pl.BlockSpec(memory_space=pltpu.VMEM),
        scratch_shapes=[
            pltpu.VMEM((n_slots, mc, n), WIRE_DTYPE),
            pltpu.VMEM((n_slots, mc, n), WIRE_DTYPE),
            pltpu.SemaphoreType.DMA((n_slots,)),
            pltpu.SemaphoreType.DMA((n_slots,)),
        ],
        compiler_params=pltpu.CompilerParams(collective_id=0),
    )(x, W1, W2)
